# baseline (speedup 1.0000x reference)
import functools

import jax
import jax.numpy as jnp
from jax import lax
from jax.experimental import pallas as pl
from jax.experimental.pallas import tpu as pltpu
from jax.experimental.pallas import tpu_sc as plsc

N = 10000
E = 160000
IN_FEAT = 256
OUT_FEAT = 256
H = 4
R = 32

TC_BLK = 1280
TC_GRID = E // TC_BLK

NC = 2
NS = 16
NW = NC * NS
ROWS = 320
NPAD = NW * ROWS
GK = 64
CH = 2048
NCHUNK = E // CH
REM = E - NCHUNK * CH
CAP = 192


def _tc_body(rel_ref, e_ref, w_ref, a_ref, out_ref):
    relv = rel_ref[0, 0, :]
    oh = (relv[:, None] == lax.broadcasted_iota(jnp.int32, (TC_BLK, R), 1))
    oh = oh.astype(jnp.bfloat16)
    fe = jnp.dot(e_ref[...], w_ref[...], preferred_element_type=jnp.float32)
    w = jnp.dot(oh, a_ref[...], preferred_element_type=jnp.float32)
    t = w * fe
    t = jnp.where(t >= 0, t, 0.2 * t)
    out_ref[...] = (t[:, 0:OUT_FEAT] + t[:, OUT_FEAT:2 * OUT_FEAT]
                    + t[:, 2 * OUT_FEAT:3 * OUT_FEAT]
                    + t[:, 3 * OUT_FEAT:4 * OUT_FEAT])


def _edge_feat(e, W_fc, attn, rel):
    rel3 = rel.reshape(TC_GRID, 1, TC_BLK)
    attn2d = attn.reshape(R, H * OUT_FEAT)
    return pl.pallas_call(
        _tc_body,
        grid=(TC_GRID,),
        in_specs=[
            pl.BlockSpec((1, 1, TC_BLK), lambda i: (i, 0, 0)),
            pl.BlockSpec((TC_BLK, IN_FEAT), lambda i: (i, 0)),
            pl.BlockSpec((IN_FEAT, H * OUT_FEAT), lambda i: (0, 0)),
            pl.BlockSpec((R, H * OUT_FEAT), lambda i: (0, 0)),
        ],
        out_specs=pl.BlockSpec((TC_BLK, OUT_FEAT), lambda i: (i, 0)),
        out_shape=jax.ShapeDtypeStruct((E, OUT_FEAT), jnp.float32),
    )(rel3, e, W_fc, attn2d)


def _sc_body(src_hbm, dst_hbm, x_hbm, feat_hbm, out_hbm,
             dstb, srcb, dstb1, srcb1, locl, srcl, eidl, msg, featb, acc,
             semA, semB, g1, g2):
    c = lax.axis_index("c")
    s = lax.axis_index("s")
    w = c * NS + s
    lo = w * ROWS

    iota16 = lax.broadcasted_iota(jnp.int32, (16,), 0)
    zeros16 = jnp.zeros((16,), jnp.float32)

    def zrow(r, carry):
        for j in range(OUT_FEAT // 16):
            acc[pl.ds(r * OUT_FEAT + j * 16, 16)] = zeros16
        return carry
    lax.fori_loop(0, ROWS + 1, zrow, 0)

    def accumulate():
        def arow(r, carry):
            rv = jnp.zeros((16,), jnp.int32) + r
            locb = plsc.load_gather(locl, [rv]) * OUT_FEAT
            for j in range(OUT_FEAT // 16):
                idx = locb + (j * 16) + iota16
                a = plsc.load_gather(acc, [idx])
                cs = pl.ds(j * 16, 16)
                plsc.store_scatter(acc, [idx], a + msg[r, cs] + featb[r, cs])
            return carry
        lax.fori_loop(0, GK, arow, 0)

    def fire():
        pltpu.async_copy(x_hbm.at[srcl.at[pl.ds(0, GK)]], msg, g1)
        pltpu.async_copy(feat_hbm.at[eidl.at[pl.ds(0, GK)]], featb, g2)

    def complete():
        pltpu.make_async_copy(x_hbm.at[srcl.at[pl.ds(0, GK)]], msg, g1).wait()
        pltpu.make_async_copy(feat_hbm.at[eidl.at[pl.ds(0, GK)]],
                              featb, g2).wait()
        accumulate()
        for l in (locl, srcl, eidl):
            for t in range(GK // 16 + 2):
                g = l[pl.ds(GK + t * 16, 16)]
                l[pl.ds(t * 16, 16)] = g

    def compress_group(dstb, srcb, base, gi, off):
        d = dstb[pl.ds(gi * 16, 16)]
        sv = srcb[pl.ds(gi * 16, 16)]
        loc = d - lo
        m = (loc >= 0) & (loc < ROWS)
        pos = off + jnp.cumsum(jnp.where(m, 1, 0)) - 1
        plsc.store_scatter(locl, [pos], loc, mask=m)
        plsc.store_scatter(srcl, [pos], sv, mask=m)
        plsc.store_scatter(eidl, [pos], base + iota16, mask=m)
        return off + plsc.all_reduce_population_count(m)

    def scan_groups(dstb, srcb, base, off, nedges):
        def group(g, off):
            prev = off
            off = compress_group(dstb, srcb, base + g * 16, g, off)

            @pl.when(jnp.all((prev < GK) & (off >= GK)))
            def _():
                fire()

            done = off >= 2 * GK

            @pl.when(jnp.all(done))
            def _():
                complete()
                fire()

            return jnp.where(done, off - GK, off)

        return lax.fori_loop(0, nedges // 16, group, off)

    def start_load(ci, dbuf, sbuf, sem):
        base = ci * CH
        pltpu.async_copy(dst_hbm.at[pl.ds(base, CH)], dbuf, sem)
        pltpu.async_copy(src_hbm.at[pl.ds(base, CH)], sbuf, sem)

    def wait_load(dbuf, sbuf, sem):
        pltpu.make_async_copy(dst_hbm.at[pl.ds(0, CH)], dbuf, sem).wait()
        pltpu.make_async_copy(src_hbm.at[pl.ds(0, CH)], sbuf, sem).wait()

    NPAIR = NCHUNK // 2
    start_load(0, dstb, srcb, semA)
    start_load(1, dstb1, srcb1, semB)

    def pair(k, off):
        wait_load(dstb, srcb, semA)
        off = scan_groups(dstb, srcb, (2 * k) * CH, off, CH)

        @pl.when(k < NPAIR - 1)
        def _():
            start_load(2 * k + 2, dstb, srcb, semA)

        wait_load(dstb1, srcb1, semB)
        off = scan_groups(dstb1, srcb1, (2 * k + 1) * CH, off, CH)

        @pl.when(k < NPAIR - 1)
        def _():
            start_load(2 * k + 3, dstb1, srcb1, semB)

        return off

    off0 = jnp.zeros((16,), jnp.int32)
    off = lax.fori_loop(0, NPAIR, pair, off0)

    pltpu.sync_copy(dst_hbm.at[pl.ds(NCHUNK * CH, REM)], dstb.at[pl.ds(0, REM)])
    pltpu.sync_copy(src_hbm.at[pl.ds(NCHUNK * CH, REM)], srcb.at[pl.ds(0, REM)])
    off = scan_groups(dstb, srcb, NCHUNK * CH, off, REM)

    @pl.when(jnp.all(off >= GK))
    def _():
        complete()
    off = jnp.where(off >= GK, off - GK, off)

    for g in range(GK // 16):
        valid = (g * 16 + iota16) < off
        for l, pad in ((srcl, 0), (eidl, 0), (locl, ROWS)):
            v = l[pl.ds(g * 16, 16)]
            l[pl.ds(g * 16, 16)] = jnp.where(valid, v, pad)
    fire()
    complete()

    pltpu.sync_copy(acc.at[pl.ds(0, ROWS * OUT_FEAT)],
                    out_hbm.at[pl.ds(lo * OUT_FEAT, ROWS * OUT_FEAT)])


@functools.cache
def _sc_scatter():
    return pl.kernel(
        _sc_body,
        out_type=jax.ShapeDtypeStruct((NPAD * OUT_FEAT,), jnp.float32),
        mesh=plsc.VectorSubcoreMesh(core_axis_name="c", subcore_axis_name="s",
                                    num_cores=NC, num_subcores=NS),
        compiler_params=pltpu.CompilerParams(needs_layout_passes=False),
        scratch_types=[
            pltpu.VMEM((CH,), jnp.int32),
            pltpu.VMEM((CH,), jnp.int32),
            pltpu.VMEM((CH,), jnp.int32),
            pltpu.VMEM((CH,), jnp.int32),
            pltpu.VMEM((CAP,), jnp.int32),
            pltpu.VMEM((CAP,), jnp.int32),
            pltpu.VMEM((CAP,), jnp.int32),
            pltpu.VMEM((GK, OUT_FEAT), jnp.float32),
            pltpu.VMEM((GK, OUT_FEAT), jnp.float32),
            pltpu.VMEM(((ROWS + 1) * OUT_FEAT,), jnp.float32),
            pltpu.SemaphoreType.DMA,
            pltpu.SemaphoreType.DMA,
            pltpu.SemaphoreType.DMA,
            pltpu.SemaphoreType.DMA,
        ],
    )


def kernel(x, e, W_fc, attn, edge_index, rel):
    src = edge_index[0].astype(jnp.int32)
    dst = edge_index[1].astype(jnp.int32)
    feat = _edge_feat(e.astype(jnp.bfloat16), W_fc.astype(jnp.bfloat16),
                      attn.astype(jnp.bfloat16), rel.astype(jnp.int32))
    out = _sc_scatter()(src, dst, x, feat)
    return out.reshape(NPAD, OUT_FEAT)[:N]


# speedup vs baseline: 1.1320x; 1.1320x over previous
import functools

import jax
import jax.numpy as jnp
from jax import lax
from jax.experimental import pallas as pl
from jax.experimental.pallas import tpu as pltpu
from jax.experimental.pallas import tpu_sc as plsc

N = 10000
E = 160000
IN_FEAT = 256
OUT_FEAT = 256
H = 4
R = 32

TC_BLK = 1280
TC_GRID = E // TC_BLK

NC = 2
NS = 16
NW = NC * NS
ROWS = 320
NPAD = NW * ROWS
GK = 64
CH = 2048
NCHUNK = E // CH
REM = E - NCHUNK * CH
CAP = 96


def _tc_body(rel_ref, e_ref, w_ref, a_ref, out_ref):
    relv = rel_ref[0, 0, :]
    oh = (relv[:, None] == lax.broadcasted_iota(jnp.int32, (TC_BLK, R), 1))
    oh = oh.astype(jnp.bfloat16)
    fe = jnp.dot(e_ref[...], w_ref[...], preferred_element_type=jnp.float32)
    w = jnp.dot(oh, a_ref[...], preferred_element_type=jnp.float32)
    t = w * fe
    t = jnp.where(t >= 0, t, 0.2 * t)
    out_ref[...] = (t[:, 0:OUT_FEAT] + t[:, OUT_FEAT:2 * OUT_FEAT]
                    + t[:, 2 * OUT_FEAT:3 * OUT_FEAT]
                    + t[:, 3 * OUT_FEAT:4 * OUT_FEAT])


def _edge_feat(e, W_fc, attn, rel):
    rel3 = rel.reshape(TC_GRID, 1, TC_BLK)
    attn2d = attn.reshape(R, H * OUT_FEAT)
    return pl.pallas_call(
        _tc_body,
        grid=(TC_GRID,),
        in_specs=[
            pl.BlockSpec((1, 1, TC_BLK), lambda i: (i, 0, 0)),
            pl.BlockSpec((TC_BLK, IN_FEAT), lambda i: (i, 0)),
            pl.BlockSpec((IN_FEAT, H * OUT_FEAT), lambda i: (0, 0)),
            pl.BlockSpec((R, H * OUT_FEAT), lambda i: (0, 0)),
        ],
        out_specs=pl.BlockSpec((TC_BLK, OUT_FEAT), lambda i: (i, 0)),
        out_shape=jax.ShapeDtypeStruct((E, OUT_FEAT), jnp.float32),
    )(rel3, e, W_fc, attn2d)


def _sc_body(src_hbm, dst_hbm, x_hbm, feat_hbm, out_hbm,
             dstb, srcb, dstb1, srcb1, locl, srcl, eidl, msg, featb, acc,
             semA, semB, g1, g2):
    c = lax.axis_index("c")
    s = lax.axis_index("s")
    w = c * NS + s
    lo = w * ROWS

    iota16 = lax.broadcasted_iota(jnp.int32, (16,), 0)
    zeros16 = jnp.zeros((16,), jnp.float32)

    def zrow(r, carry):
        for j in range(OUT_FEAT // 16):
            acc[pl.ds(r * OUT_FEAT + j * 16, 16)] = zeros16
        return carry
    lax.fori_loop(0, ROWS + 1, zrow, 0)

    def accumulate():
        def arow(r4, carry):
            for u in range(4):
                r = r4 * 4 + u
                rv = jnp.zeros((16,), jnp.int32) + r
                locb = plsc.load_gather(locl, [rv]) * OUT_FEAT
                for j in range(OUT_FEAT // 16):
                    idx = locb + (j * 16) + iota16
                    cs = pl.ds(j * 16, 16)
                    plsc.addupdate_scatter(acc, [idx], msg[r, cs] + featb[r, cs])
            return carry
        lax.fori_loop(0, GK // 4, arow, 0)

    def drain128():
        cp1 = pltpu.async_copy(x_hbm.at[srcl.at[pl.ds(0, GK)]], msg, g1)
        cp2 = pltpu.async_copy(feat_hbm.at[eidl.at[pl.ds(0, GK)]], featb, g2)
        cp1.wait()
        cp2.wait()
        accumulate()
        for l in (locl, srcl, eidl):
            g = l[pl.ds(GK, 16)]
            l[pl.ds(0, 16)] = g

    def compress_group(dstb, srcb, base, gi, off):
        d = dstb[pl.ds(gi * 16, 16)]
        sv = srcb[pl.ds(gi * 16, 16)]
        loc = d - lo
        m = (loc >= 0) & (loc < ROWS)
        pos = off + jnp.cumsum(jnp.where(m, 1, 0)) - 1
        plsc.store_scatter(locl, [pos], loc, mask=m)
        plsc.store_scatter(srcl, [pos], sv, mask=m)
        plsc.store_scatter(eidl, [pos], base + iota16, mask=m)
        return off + plsc.all_reduce_population_count(m)

    def scan_groups(dstb, srcb, base, off, nedges):
        def group(g, off):
            off = compress_group(dstb, srcb, base + g * 16, g, off)
            full = off >= GK
            do_drain = jnp.all(full)

            @pl.when(do_drain)
            def _():
                drain128()

            return jnp.where(full, off - GK, off)

        return lax.fori_loop(0, nedges // 16, group, off)

    def start_load(ci, dbuf, sbuf, sem):
        base = ci * CH
        pltpu.async_copy(dst_hbm.at[pl.ds(base, CH)], dbuf, sem)
        pltpu.async_copy(src_hbm.at[pl.ds(base, CH)], sbuf, sem)

    def wait_load(dbuf, sbuf, sem):
        pltpu.make_async_copy(dst_hbm.at[pl.ds(0, CH)], dbuf, sem).wait()
        pltpu.make_async_copy(src_hbm.at[pl.ds(0, CH)], sbuf, sem).wait()

    NPAIR = NCHUNK // 2
    start_load(0, dstb, srcb, semA)
    start_load(1, dstb1, srcb1, semB)

    def pair(k, off):
        wait_load(dstb, srcb, semA)
        off = scan_groups(dstb, srcb, (2 * k) * CH, off, CH)

        @pl.when(k < NPAIR - 1)
        def _():
            start_load(2 * k + 2, dstb, srcb, semA)

        wait_load(dstb1, srcb1, semB)
        off = scan_groups(dstb1, srcb1, (2 * k + 1) * CH, off, CH)

        @pl.when(k < NPAIR - 1)
        def _():
            start_load(2 * k + 3, dstb1, srcb1, semB)

        return off

    off0 = jnp.zeros((16,), jnp.int32)
    off = lax.fori_loop(0, NPAIR, pair, off0)

    pltpu.sync_copy(dst_hbm.at[pl.ds(NCHUNK * CH, REM)], dstb.at[pl.ds(0, REM)])
    pltpu.sync_copy(src_hbm.at[pl.ds(NCHUNK * CH, REM)], srcb.at[pl.ds(0, REM)])
    off = scan_groups(dstb, srcb, NCHUNK * CH, off, REM)

    for g in range(GK // 16):
        valid = (g * 16 + iota16) < off
        for l, pad in ((srcl, 0), (eidl, 0), (locl, ROWS)):
            v = l[pl.ds(g * 16, 16)]
            l[pl.ds(g * 16, 16)] = jnp.where(valid, v, pad)
    drain128()

    pltpu.sync_copy(acc.at[pl.ds(0, ROWS * OUT_FEAT)],
                    out_hbm.at[pl.ds(lo * OUT_FEAT, ROWS * OUT_FEAT)])


@functools.cache
def _sc_scatter():
    return pl.kernel(
        _sc_body,
        out_type=jax.ShapeDtypeStruct((NPAD * OUT_FEAT,), jnp.float32),
        mesh=plsc.VectorSubcoreMesh(core_axis_name="c", subcore_axis_name="s",
                                    num_cores=NC, num_subcores=NS),
        compiler_params=pltpu.CompilerParams(needs_layout_passes=False),
        scratch_types=[
            pltpu.VMEM((CH,), jnp.int32),
            pltpu.VMEM((CH,), jnp.int32),
            pltpu.VMEM((CH,), jnp.int32),
            pltpu.VMEM((CH,), jnp.int32),
            pltpu.VMEM((CAP,), jnp.int32),
            pltpu.VMEM((CAP,), jnp.int32),
            pltpu.VMEM((CAP,), jnp.int32),
            pltpu.VMEM((GK, OUT_FEAT), jnp.float32),
            pltpu.VMEM((GK, OUT_FEAT), jnp.float32),
            pltpu.VMEM(((ROWS + 1) * OUT_FEAT,), jnp.float32),
            pltpu.SemaphoreType.DMA,
            pltpu.SemaphoreType.DMA,
            pltpu.SemaphoreType.DMA,
            pltpu.SemaphoreType.DMA,
        ],
    )


def kernel(x, e, W_fc, attn, edge_index, rel):
    src = edge_index[0].astype(jnp.int32)
    dst = edge_index[1].astype(jnp.int32)
    feat = _edge_feat(e.astype(jnp.bfloat16), W_fc.astype(jnp.bfloat16),
                      attn.astype(jnp.bfloat16), rel.astype(jnp.int32))
    out = _sc_scatter()(src, dst, x, feat)
    return out.reshape(NPAD, OUT_FEAT)[:N]


# speedup vs baseline: 1.1957x; 1.0563x over previous
import functools

import jax
import jax.numpy as jnp
from jax import lax
from jax.experimental import pallas as pl
from jax.experimental.pallas import tpu as pltpu
from jax.experimental.pallas import tpu_sc as plsc

N = 10000
E = 160000
IN_FEAT = 256
OUT_FEAT = 256
H = 4
R = 32

TC_BLK = 1280
TC_GRID = E // TC_BLK

NC = 2
NS = 16
NW = NC * NS
ROWS = 320
NPAD = NW * ROWS
GK = 64
CH = 2048
NCHUNK = E // CH
REM = E - NCHUNK * CH
CAP = 96


def _tc_body(rel_ref, e_ref, w_ref, a_ref, out_ref):
    relv = rel_ref[0, 0, :]
    oh = (relv[:, None] == lax.broadcasted_iota(jnp.int32, (TC_BLK, R), 1))
    oh = oh.astype(jnp.bfloat16)
    fe = jnp.dot(e_ref[...].astype(jnp.bfloat16), w_ref[...],
                 preferred_element_type=jnp.float32)
    w = jnp.dot(oh, a_ref[...], preferred_element_type=jnp.float32)
    t = w * fe
    t = jnp.where(t >= 0, t, 0.2 * t)
    out_ref[...] = (t[:, 0:OUT_FEAT] + t[:, OUT_FEAT:2 * OUT_FEAT]
                    + t[:, 2 * OUT_FEAT:3 * OUT_FEAT]
                    + t[:, 3 * OUT_FEAT:4 * OUT_FEAT])


def _edge_feat(e, W_fc, attn, rel):
    rel3 = rel.reshape(TC_GRID, 1, TC_BLK)
    attn2d = attn.reshape(R, H * OUT_FEAT)
    return pl.pallas_call(
        _tc_body,
        grid=(TC_GRID,),
        in_specs=[
            pl.BlockSpec((1, 1, TC_BLK), lambda i: (i, 0, 0)),
            pl.BlockSpec((TC_BLK, IN_FEAT), lambda i: (i, 0)),
            pl.BlockSpec((IN_FEAT, H * OUT_FEAT), lambda i: (0, 0)),
            pl.BlockSpec((R, H * OUT_FEAT), lambda i: (0, 0)),
        ],
        out_specs=pl.BlockSpec((TC_BLK, OUT_FEAT), lambda i: (i, 0)),
        out_shape=jax.ShapeDtypeStruct((E, OUT_FEAT), jnp.float32),
    )(rel3, e, W_fc, attn2d)


def _sc_body(src_hbm, dst_hbm, x_hbm, feat_hbm, out_hbm,
             dstb, srcb, dstb1, srcb1, locl, srcl, eidl, msg, featb, acc,
             semA, semB, g1, g2):
    c = lax.axis_index("c")
    s = lax.axis_index("s")
    w = c * NS + s
    lo = w * ROWS

    iota16 = lax.broadcasted_iota(jnp.int32, (16,), 0)
    zeros16 = jnp.zeros((16,), jnp.float32)

    def zrow(r, carry):
        for j in range(OUT_FEAT // 16):
            acc[pl.ds(r * OUT_FEAT + j * 16, 16)] = zeros16
        return carry
    lax.fori_loop(0, ROWS + 1, zrow, 0)

    def accumulate():
        def arow(r4, carry):
            for u in range(4):
                r = r4 * 4 + u
                rv = jnp.zeros((16,), jnp.int32) + r
                locb = plsc.load_gather(locl, [rv]) * OUT_FEAT
                for j in range(OUT_FEAT // 16):
                    idx = locb + (j * 16) + iota16
                    cs = pl.ds(j * 16, 16)
                    plsc.addupdate_scatter(acc, [idx], msg[r, cs] + featb[r, cs])
            return carry
        lax.fori_loop(0, GK // 4, arow, 0)

    def drain128():
        cp1 = pltpu.async_copy(x_hbm.at[srcl.at[pl.ds(0, GK)]], msg, g1)
        cp2 = pltpu.async_copy(feat_hbm.at[eidl.at[pl.ds(0, GK)]], featb, g2)
        cp1.wait()
        cp2.wait()
        accumulate()
        for l in (locl, srcl, eidl):
            g = l[pl.ds(GK, 16)]
            l[pl.ds(0, 16)] = g

    def compress_group(dstb, srcb, base, gi, off):
        d = dstb[pl.ds(gi * 16, 16)]
        sv = srcb[pl.ds(gi * 16, 16)]
        loc = d - lo
        m = (loc >= 0) & (loc < ROWS)
        pos = off + jnp.cumsum(jnp.where(m, 1, 0)) - 1
        plsc.store_scatter(locl, [pos], loc, mask=m)
        plsc.store_scatter(srcl, [pos], sv, mask=m)
        plsc.store_scatter(eidl, [pos], base + iota16, mask=m)
        return off + plsc.all_reduce_population_count(m)

    def scan_groups(dstb, srcb, base, off, nedges):
        def group(g, off):
            off = compress_group(dstb, srcb, base + g * 16, g, off)
            full = off >= GK
            do_drain = jnp.all(full)

            @pl.when(do_drain)
            def _():
                drain128()

            return jnp.where(full, off - GK, off)

        return lax.fori_loop(0, nedges // 16, group, off)

    def start_load(ci, dbuf, sbuf, sem):
        base = ci * CH
        pltpu.async_copy(dst_hbm.at[pl.ds(base, CH)], dbuf, sem)
        pltpu.async_copy(src_hbm.at[pl.ds(base, CH)], sbuf, sem)

    def wait_load(dbuf, sbuf, sem):
        pltpu.make_async_copy(dst_hbm.at[pl.ds(0, CH)], dbuf, sem).wait()
        pltpu.make_async_copy(src_hbm.at[pl.ds(0, CH)], sbuf, sem).wait()

    NPAIR = NCHUNK // 2
    start_load(0, dstb, srcb, semA)
    start_load(1, dstb1, srcb1, semB)

    def pair(k, off):
        wait_load(dstb, srcb, semA)
        off = scan_groups(dstb, srcb, (2 * k) * CH, off, CH)

        @pl.when(k < NPAIR - 1)
        def _():
            start_load(2 * k + 2, dstb, srcb, semA)

        wait_load(dstb1, srcb1, semB)
        off = scan_groups(dstb1, srcb1, (2 * k + 1) * CH, off, CH)

        @pl.when(k < NPAIR - 1)
        def _():
            start_load(2 * k + 3, dstb1, srcb1, semB)

        return off

    off0 = jnp.zeros((16,), jnp.int32)
    off = lax.fori_loop(0, NPAIR, pair, off0)

    pltpu.sync_copy(dst_hbm.at[pl.ds(NCHUNK * CH, REM)], dstb.at[pl.ds(0, REM)])
    pltpu.sync_copy(src_hbm.at[pl.ds(NCHUNK * CH, REM)], srcb.at[pl.ds(0, REM)])
    off = scan_groups(dstb, srcb, NCHUNK * CH, off, REM)

    for g in range(GK // 16):
        valid = (g * 16 + iota16) < off
        for l, pad in ((srcl, 0), (eidl, 0), (locl, ROWS)):
            v = l[pl.ds(g * 16, 16)]
            l[pl.ds(g * 16, 16)] = jnp.where(valid, v, pad)
    drain128()

    pltpu.sync_copy(acc.at[pl.ds(0, ROWS * OUT_FEAT)],
                    out_hbm.at[pl.ds(lo * OUT_FEAT, ROWS * OUT_FEAT)])


@functools.cache
def _sc_scatter():
    return pl.kernel(
        _sc_body,
        out_type=jax.ShapeDtypeStruct((NPAD * OUT_FEAT,), jnp.float32),
        mesh=plsc.VectorSubcoreMesh(core_axis_name="c", subcore_axis_name="s",
                                    num_cores=NC, num_subcores=NS),
        compiler_params=pltpu.CompilerParams(needs_layout_passes=False),
        scratch_types=[
            pltpu.VMEM((CH,), jnp.int32),
            pltpu.VMEM((CH,), jnp.int32),
            pltpu.VMEM((CH,), jnp.int32),
            pltpu.VMEM((CH,), jnp.int32),
            pltpu.VMEM((CAP,), jnp.int32),
            pltpu.VMEM((CAP,), jnp.int32),
            pltpu.VMEM((CAP,), jnp.int32),
            pltpu.VMEM((GK, OUT_FEAT), jnp.float32),
            pltpu.VMEM((GK, OUT_FEAT), jnp.float32),
            pltpu.VMEM(((ROWS + 1) * OUT_FEAT,), jnp.float32),
            pltpu.SemaphoreType.DMA,
            pltpu.SemaphoreType.DMA,
            pltpu.SemaphoreType.DMA,
            pltpu.SemaphoreType.DMA,
        ],
    )


def kernel(x, e, W_fc, attn, edge_index, rel):
    src = edge_index[0].astype(jnp.int32)
    dst = edge_index[1].astype(jnp.int32)
    feat = _edge_feat(e, W_fc.astype(jnp.bfloat16),
                      attn.astype(jnp.bfloat16), rel.astype(jnp.int32))
    out = _sc_scatter()(src, dst, x, feat)
    return out.reshape(NPAD, OUT_FEAT)[:N]


# speedup vs baseline: 1.2138x; 1.0152x over previous
import functools

import jax
import jax.numpy as jnp
from jax import lax
from jax.experimental import pallas as pl
from jax.experimental.pallas import tpu as pltpu
from jax.experimental.pallas import tpu_sc as plsc

N = 10000
E = 160000
IN_FEAT = 256
OUT_FEAT = 256
H = 4
R = 32

TC_BLK = 1280
TC_GRID = E // TC_BLK

NC = 2
NS = 16
NW = NC * NS
ROWS = 320
NPAD = NW * ROWS
GK = 64
CH = 2048
NCHUNK = E // CH
REM = E - NCHUNK * CH
CAP = 96


def _tc_body(rel_ref, e_ref, w_ref, a_ref, out_ref):
    relv = rel_ref[0, 0, :]
    oh = (relv[:, None] == lax.broadcasted_iota(jnp.int32, (TC_BLK, R), 1))
    oh = oh.astype(jnp.bfloat16)
    fe = jnp.dot(e_ref[...].astype(jnp.bfloat16), w_ref[...],
                 preferred_element_type=jnp.float32)
    w = jnp.dot(oh, a_ref[...], preferred_element_type=jnp.float32)
    t = w * fe
    t = jnp.where(t >= 0, t, 0.2 * t)
    out_ref[...] = (t[:, 0:OUT_FEAT] + t[:, OUT_FEAT:2 * OUT_FEAT]
                    + t[:, 2 * OUT_FEAT:3 * OUT_FEAT]
                    + t[:, 3 * OUT_FEAT:4 * OUT_FEAT])


def _edge_feat(e, W_fc, attn, rel):
    rel3 = rel.reshape(TC_GRID, 1, TC_BLK)
    attn2d = attn.reshape(R, H * OUT_FEAT)
    return pl.pallas_call(
        _tc_body,
        grid=(TC_GRID,),
        in_specs=[
            pl.BlockSpec((1, 1, TC_BLK), lambda i: (i, 0, 0)),
            pl.BlockSpec((TC_BLK, IN_FEAT), lambda i: (i, 0)),
            pl.BlockSpec((IN_FEAT, H * OUT_FEAT), lambda i: (0, 0)),
            pl.BlockSpec((R, H * OUT_FEAT), lambda i: (0, 0)),
        ],
        out_specs=pl.BlockSpec((TC_BLK, OUT_FEAT), lambda i: (i, 0)),
        out_shape=jax.ShapeDtypeStruct((E, OUT_FEAT), jnp.float32),
    )(rel3, e, W_fc, attn2d)


def _sc_body(src_hbm, dst_hbm, x_hbm, feat_hbm, out_hbm,
             dstb, srcb, dstb1, srcb1, locl, srcl, eidl, msg, featb, acc,
             semA, semB, g1, g2):
    c = lax.axis_index("c")
    s = lax.axis_index("s")
    w = c * NS + s
    lo = w * ROWS

    iota16 = lax.broadcasted_iota(jnp.int32, (16,), 0)
    zeros16 = jnp.zeros((16,), jnp.float32)

    def zrow(r, carry):
        for j in range(OUT_FEAT // 16):
            acc[pl.ds(r * OUT_FEAT + j * 16, 16)] = zeros16
        return carry
    lax.fori_loop(0, ROWS + 1, zrow, 0)

    def accumulate():
        def arow(r4, carry):
            for u in range(4):
                r = r4 * 4 + u
                rv = jnp.zeros((16,), jnp.int32) + r
                locb = plsc.load_gather(locl, [rv]) * OUT_FEAT
                for j in range(OUT_FEAT // 16):
                    idx = locb + (j * 16) + iota16
                    cs = pl.ds(j * 16, 16)
                    plsc.addupdate_scatter(acc, [idx], msg[r, cs] + featb[r, cs])
            return carry
        lax.fori_loop(0, GK // 4, arow, 0)

    def drain128():
        cp1 = pltpu.async_copy(x_hbm.at[srcl.at[pl.ds(0, GK)]], msg, g1)
        cp2 = pltpu.async_copy(feat_hbm.at[eidl.at[pl.ds(0, GK)]], featb, g2)
        cp1.wait()
        cp2.wait()
        accumulate()
        for l in (locl, srcl, eidl):
            g = l[pl.ds(GK, 16)]
            l[pl.ds(0, 16)] = g

    def compress_group(dstb, srcb, base, gi, off):
        d = dstb[pl.ds(gi * 16, 16)]
        sv = srcb[pl.ds(gi * 16, 16)]
        loc = d - lo
        m = (loc >= 0) & (loc < ROWS)
        pos = off + jnp.cumsum(jnp.where(m, 1, 0)) - 1
        plsc.store_scatter(locl, [pos], loc, mask=m)
        plsc.store_scatter(srcl, [pos], sv, mask=m)
        plsc.store_scatter(eidl, [pos], base + iota16, mask=m)
        return off + plsc.all_reduce_population_count(m)

    def scan_groups(dstb, srcb, base, off, nedges):
        def group(g2, off):
            for u in range(2):
                g = g2 * 2 + u
                off = compress_group(dstb, srcb, base + g * 16, g, off)
                full = off >= GK
                do_drain = jnp.all(full)

                @pl.when(do_drain)
                def _():
                    drain128()

                off = jnp.where(full, off - GK, off)
            return off

        return lax.fori_loop(0, nedges // 32, group, off)

    def start_load(ci, dbuf, sbuf, sem):
        base = ci * CH
        pltpu.async_copy(dst_hbm.at[pl.ds(base, CH)], dbuf, sem)
        pltpu.async_copy(src_hbm.at[pl.ds(base, CH)], sbuf, sem)

    def wait_load(dbuf, sbuf, sem):
        pltpu.make_async_copy(dst_hbm.at[pl.ds(0, CH)], dbuf, sem).wait()
        pltpu.make_async_copy(src_hbm.at[pl.ds(0, CH)], sbuf, sem).wait()

    NPAIR = NCHUNK // 2
    start_load(0, dstb, srcb, semA)
    start_load(1, dstb1, srcb1, semB)

    def pair(k, off):
        wait_load(dstb, srcb, semA)
        off = scan_groups(dstb, srcb, (2 * k) * CH, off, CH)

        @pl.when(k < NPAIR - 1)
        def _():
            start_load(2 * k + 2, dstb, srcb, semA)

        wait_load(dstb1, srcb1, semB)
        off = scan_groups(dstb1, srcb1, (2 * k + 1) * CH, off, CH)

        @pl.when(k < NPAIR - 1)
        def _():
            start_load(2 * k + 3, dstb1, srcb1, semB)

        return off

    off0 = jnp.zeros((16,), jnp.int32)
    off = lax.fori_loop(0, NPAIR, pair, off0)

    pltpu.sync_copy(dst_hbm.at[pl.ds(NCHUNK * CH, REM)], dstb.at[pl.ds(0, REM)])
    pltpu.sync_copy(src_hbm.at[pl.ds(NCHUNK * CH, REM)], srcb.at[pl.ds(0, REM)])
    off = scan_groups(dstb, srcb, NCHUNK * CH, off, REM)

    for g in range(GK // 16):
        valid = (g * 16 + iota16) < off
        for l, pad in ((srcl, 0), (eidl, 0), (locl, ROWS)):
            v = l[pl.ds(g * 16, 16)]
            l[pl.ds(g * 16, 16)] = jnp.where(valid, v, pad)
    drain128()

    pltpu.sync_copy(acc.at[pl.ds(0, ROWS * OUT_FEAT)],
                    out_hbm.at[pl.ds(lo * OUT_FEAT, ROWS * OUT_FEAT)])


@functools.cache
def _sc_scatter():
    return pl.kernel(
        _sc_body,
        out_type=jax.ShapeDtypeStruct((NPAD * OUT_FEAT,), jnp.float32),
        mesh=plsc.VectorSubcoreMesh(core_axis_name="c", subcore_axis_name="s",
                                    num_cores=NC, num_subcores=NS),
        compiler_params=pltpu.CompilerParams(needs_layout_passes=False),
        scratch_types=[
            pltpu.VMEM((CH,), jnp.int32),
            pltpu.VMEM((CH,), jnp.int32),
            pltpu.VMEM((CH,), jnp.int32),
            pltpu.VMEM((CH,), jnp.int32),
            pltpu.VMEM((CAP,), jnp.int32),
            pltpu.VMEM((CAP,), jnp.int32),
            pltpu.VMEM((CAP,), jnp.int32),
            pltpu.VMEM((GK, OUT_FEAT), jnp.float32),
            pltpu.VMEM((GK, OUT_FEAT), jnp.float32),
            pltpu.VMEM(((ROWS + 1) * OUT_FEAT,), jnp.float32),
            pltpu.SemaphoreType.DMA,
            pltpu.SemaphoreType.DMA,
            pltpu.SemaphoreType.DMA,
            pltpu.SemaphoreType.DMA,
        ],
    )


def kernel(x, e, W_fc, attn, edge_index, rel):
    src = edge_index[0].astype(jnp.int32)
    dst = edge_index[1].astype(jnp.int32)
    feat = _edge_feat(e, W_fc.astype(jnp.bfloat16),
                      attn.astype(jnp.bfloat16), rel.astype(jnp.int32))
    out = _sc_scatter()(src, dst, x, feat)
    return out.reshape(NPAD, OUT_FEAT)[:N]
